# initial kernel scaffold (unmeasured)
import jax
import jax.numpy as jnp
from jax import lax
from jax.experimental import pallas as pl
from jax.experimental.pallas import tpu as pltpu

N_DEV = 4
B_PER = 2
SQ = 512
HQ = 32
DH = 64
WINDOW = 128


def _allgather_weights(wq, wo):
    dq, fq = wq.shape
    fo, do = wo.shape

    def body(wq_ref, wo_ref, outq_ref, outo_ref,
             send_q, recv_q, send_o, recv_o):
        my = lax.axis_index("i")
        left = lax.rem(my - 1 + N_DEV, N_DEV)
        right = lax.rem(my + 1, N_DEV)

        outq_ref[0] = wq_ref[...]
        outo_ref[0] = wo_ref[...]

        barrier_sem = pltpu.get_barrier_semaphore()
        for nbr in (left, right):
            pl.semaphore_signal(
                barrier_sem, inc=1,
                device_id=(nbr,), device_id_type=pl.DeviceIdType.MESH,
            )
        pl.semaphore_wait(barrier_sem, 2)

        for h in range(N_DEV - 1):
            rq = pltpu.make_async_remote_copy(
                src_ref=outq_ref.at[h],
                dst_ref=outq_ref.at[h + 1],
                send_sem=send_q.at[h],
                recv_sem=recv_q.at[h],
                device_id=(right,),
                device_id_type=pl.DeviceIdType.MESH,
            )
            ro = pltpu.make_async_remote_copy(
                src_ref=outo_ref.at[h],
                dst_ref=outo_ref.at[h + 1],
                send_sem=send_o.at[h],
                recv_sem=recv_o.at[h],
                device_id=(right,),
                device_id_type=pl.DeviceIdType.MESH,
            )
            rq.start()
            ro.start()
            rq.wait()
            ro.wait()

    return pl.pallas_call(
        body,
        out_shape=(
            jax.ShapeDtypeStruct((N_DEV, dq, fq), wq.dtype),
            jax.ShapeDtypeStruct((N_DEV, fo, do), wo.dtype),
        ),
        in_specs=[
            pl.BlockSpec(memory_space=pltpu.VMEM),
            pl.BlockSpec(memory_space=pltpu.VMEM),
        ],
        out_specs=(
            pl.BlockSpec(memory_space=pltpu.VMEM),
            pl.BlockSpec(memory_space=pltpu.VMEM),
        ),
        scratch_shapes=[
            pltpu.SemaphoreType.DMA((N_DEV - 1,)),
            pltpu.SemaphoreType.DMA((N_DEV - 1,)),
            pltpu.SemaphoreType.DMA((N_DEV - 1,)),
            pltpu.SemaphoreType.DMA((N_DEV - 1,)),
        ],
        compiler_params=pltpu.CompilerParams(collective_id=0),
    )(wq, wo)


def kernel(x, Wq, K_ext, V_ext, Wo):
    my = lax.axis_index("i")

    x_bf = x.astype(jnp.bfloat16)
    ag_q, ag_o = _allgather_weights(
        Wq.astype(jnp.bfloat16), Wo.astype(jnp.bfloat16)
    )

    perm = (my - jnp.arange(N_DEV)) % N_DEV
    wq_g = jnp.take(ag_q, perm, axis=0)
    wo_g = jnp.take(ag_o, perm, axis=0)

    k = lax.dynamic_slice_in_dim(K_ext, my * B_PER, B_PER, axis=0)
    v = lax.dynamic_slice_in_dim(V_ext, my * B_PER, B_PER, axis=0)
    k = k.astype(jnp.bfloat16)
    v = v.astype(jnp.bfloat16)

    q = jnp.einsum("bsd,gdf->bsgf", x_bf, wq_g,
                   preferred_element_type=jnp.bfloat16)
    q = q.reshape(B_PER, SQ, HQ, DH)

    scores = jnp.einsum("bihd,bjhd->bhij", q, k,
                        preferred_element_type=jnp.float32) * 0.125
    idx = jnp.arange(SQ)
    mask = jnp.abs(idx[:, None] - idx[None, :]) <= WINDOW
    scores = jnp.where(mask[None, None, :, :], scores, -1e9)
    scores = scores - scores.max(axis=-1, keepdims=True)
    w = jnp.exp(scores)
    w = w / w.sum(axis=-1, keepdims=True)

    ctx = jnp.einsum("bhij,bjhd->bihd", w.astype(jnp.bfloat16), v,
                     preferred_element_type=jnp.bfloat16)
    ctx = ctx.reshape(B_PER, SQ, N_DEV, HQ // N_DEV * DH)

    out = jnp.einsum("bsgf,gfe->bse", ctx, wo_g,
                     preferred_element_type=jnp.float32)
    return out.astype(jnp.float32)


# baseline (device time: 158450 ns/iter reference)
import jax
import jax.numpy as jnp
from jax import lax
from jax.experimental import pallas as pl
from jax.experimental.pallas import tpu as pltpu

N_DEV = 4
B_PER = 2
SQ = 512
HQ = 32
DH = 64
WINDOW = 128


def _allgather_weights(wq, wo):
    dq, fq = wq.shape
    fo, do = wo.shape

    def body(wq_ref, wo_ref, outq_ref, outo_ref,
             send_q, recv_q, send_o, recv_o):
        my = lax.axis_index("i")
        left = lax.rem(my - 1 + N_DEV, N_DEV)
        right = lax.rem(my + 1, N_DEV)

        outq_ref[0] = wq_ref[...]
        outo_ref[0] = wo_ref[...]

        barrier_sem = pltpu.get_barrier_semaphore()
        for nbr in (left, right):
            pl.semaphore_signal(
                barrier_sem, inc=1,
                device_id=(nbr,), device_id_type=pl.DeviceIdType.MESH,
            )
        pl.semaphore_wait(barrier_sem, 2)

        for h in range(N_DEV - 1):
            rq = pltpu.make_async_remote_copy(
                src_ref=outq_ref.at[h],
                dst_ref=outq_ref.at[h + 1],
                send_sem=send_q.at[h],
                recv_sem=recv_q.at[h],
                device_id=(right,),
                device_id_type=pl.DeviceIdType.MESH,
            )
            ro = pltpu.make_async_remote_copy(
                src_ref=outo_ref.at[h],
                dst_ref=outo_ref.at[h + 1],
                send_sem=send_o.at[h],
                recv_sem=recv_o.at[h],
                device_id=(right,),
                device_id_type=pl.DeviceIdType.MESH,
            )
            rq.start()
            ro.start()
            rq.wait()
            ro.wait()

    return pl.pallas_call(
        body,
        out_shape=(
            jax.ShapeDtypeStruct((N_DEV, dq, fq), wq.dtype),
            jax.ShapeDtypeStruct((N_DEV, fo, do), wo.dtype),
        ),
        in_specs=[
            pl.BlockSpec(memory_space=pltpu.VMEM),
            pl.BlockSpec(memory_space=pltpu.VMEM),
        ],
        out_specs=(
            pl.BlockSpec(memory_space=pltpu.VMEM),
            pl.BlockSpec(memory_space=pltpu.VMEM),
        ),
        scratch_shapes=[
            pltpu.SemaphoreType.DMA((N_DEV - 1,)),
            pltpu.SemaphoreType.DMA((N_DEV - 1,)),
            pltpu.SemaphoreType.DMA((N_DEV - 1,)),
            pltpu.SemaphoreType.DMA((N_DEV - 1,)),
        ],
        compiler_params=pltpu.CompilerParams(collective_id=0),
    )(wq, wo)


def _attention(q, k, v):
    n = q.shape[0]

    def body(q_ref, k_ref, v_ref, o_ref):
        qm = q_ref[0] * 0.125
        km = k_ref[0]
        s = lax.dot_general(
            qm, km, (((1,), (1,)), ((), ())),
            preferred_element_type=jnp.float32,
        )
        i = lax.broadcasted_iota(jnp.int32, (SQ, SQ), 0)
        j = lax.broadcasted_iota(jnp.int32, (SQ, SQ), 1)
        s = jnp.where(jnp.abs(i - j) <= WINDOW, s, -1e9)
        s = s - jnp.max(s, axis=-1, keepdims=True)
        w = jnp.exp(s)
        w = (w / jnp.sum(w, axis=-1, keepdims=True)).astype(jnp.bfloat16)
        o_ref[0] = lax.dot_general(
            w, v_ref[0], (((1,), (0,)), ((), ())),
            preferred_element_type=jnp.float32,
        ).astype(jnp.bfloat16)

    spec = pl.BlockSpec((1, SQ, DH), lambda i: (i, 0, 0))
    return pl.pallas_call(
        body,
        grid=(n,),
        in_specs=[spec, spec, spec],
        out_specs=spec,
        out_shape=jax.ShapeDtypeStruct((n, SQ, DH), jnp.bfloat16),
    )(q, k, v)


def kernel(x, Wq, K_ext, V_ext, Wo):
    my = lax.axis_index("i")

    x_bf = x.astype(jnp.bfloat16)
    ag_q, ag_o = _allgather_weights(
        Wq.astype(jnp.bfloat16), Wo.astype(jnp.bfloat16)
    )

    perm = (my - jnp.arange(N_DEV)) % N_DEV
    wq_g = jnp.take(ag_q, perm, axis=0)
    wo_g = jnp.take(ag_o, perm, axis=0)

    k = lax.dynamic_slice_in_dim(K_ext, my * B_PER, B_PER, axis=0)
    v = lax.dynamic_slice_in_dim(V_ext, my * B_PER, B_PER, axis=0)
    k = k.astype(jnp.bfloat16).transpose(0, 2, 1, 3).reshape(B_PER * HQ, SQ, DH)
    v = v.astype(jnp.bfloat16).transpose(0, 2, 1, 3).reshape(B_PER * HQ, SQ, DH)

    r = HQ // N_DEV
    wq5 = wq_g.reshape(N_DEV, x.shape[-1], r, DH)
    q = jnp.einsum("bsm,gmrd->bgrsd", x_bf, wq5,
                   preferred_element_type=jnp.bfloat16)
    q = q.reshape(B_PER * HQ, SQ, DH)

    ctx = _attention(q, k, v)
    ctx = ctx.reshape(B_PER, N_DEV, r, SQ, DH)

    wo5 = wo_g.reshape(N_DEV, r, DH, Wo.shape[-1])
    out = jnp.einsum("bgrsd,grde->bse", ctx, wo5,
                     preferred_element_type=jnp.float32)
    return out.astype(jnp.float32)


# device time: 135883 ns/iter; 1.1661x vs baseline; 1.1661x over previous
import jax
import jax.numpy as jnp
from jax import lax
from jax.experimental import pallas as pl
from jax.experimental.pallas import tpu as pltpu

N_DEV = 4
B_PER = 2
SQ = 512
HQ = 32
DH = 64
WINDOW = 128


def _allgather_weights(wq, wo):
    dq, fq = wq.shape
    fo, do = wo.shape

    def body(wq_ref, wo_ref, outq_ref, outo_ref,
             send_q, recv_q, send_o, recv_o):
        my = lax.axis_index("i")
        left = lax.rem(my - 1 + N_DEV, N_DEV)
        right = lax.rem(my + 1, N_DEV)

        outq_ref[0] = wq_ref[...]
        outo_ref[0] = wo_ref[...]

        barrier_sem = pltpu.get_barrier_semaphore()
        for nbr in (left, right):
            pl.semaphore_signal(
                barrier_sem, inc=1,
                device_id=(nbr,), device_id_type=pl.DeviceIdType.MESH,
            )
        pl.semaphore_wait(barrier_sem, 2)

        def send(out_ref, s_sems, r_sems, src_slot, dst_slot, sem, dst):
            return pltpu.make_async_remote_copy(
                src_ref=out_ref.at[src_slot],
                dst_ref=out_ref.at[dst_slot],
                send_sem=s_sems.at[sem],
                recv_sem=r_sems.at[sem],
                device_id=(dst,),
                device_id_type=pl.DeviceIdType.MESH,
            )

        r0 = [
            send(outq_ref, send_q, recv_q, 0, 1, 0, right),
            send(outo_ref, send_o, recv_o, 0, 1, 0, right),
            send(outq_ref, send_q, recv_q, 0, 3, 2, left),
            send(outo_ref, send_o, recv_o, 0, 3, 2, left),
        ]
        for r in r0:
            r.start()
        for r in r0:
            r.wait()
        r1 = [
            send(outq_ref, send_q, recv_q, 1, 2, 1, right),
            send(outo_ref, send_o, recv_o, 1, 2, 1, right),
        ]
        for r in r1:
            r.start()
        for r in r1:
            r.wait()

    return pl.pallas_call(
        body,
        out_shape=(
            jax.ShapeDtypeStruct((N_DEV, dq, fq), wq.dtype),
            jax.ShapeDtypeStruct((N_DEV, fo, do), wo.dtype),
        ),
        in_specs=[
            pl.BlockSpec(memory_space=pltpu.VMEM),
            pl.BlockSpec(memory_space=pltpu.VMEM),
        ],
        out_specs=(
            pl.BlockSpec(memory_space=pltpu.VMEM),
            pl.BlockSpec(memory_space=pltpu.VMEM),
        ),
        scratch_shapes=[
            pltpu.SemaphoreType.DMA((N_DEV - 1,)),
            pltpu.SemaphoreType.DMA((N_DEV - 1,)),
            pltpu.SemaphoreType.DMA((N_DEV - 1,)),
            pltpu.SemaphoreType.DMA((N_DEV - 1,)),
        ],
        compiler_params=pltpu.CompilerParams(collective_id=0),
    )(wq, wo)


def _attention(q, k, v):
    n = q.shape[0]

    def body(q_ref, k_ref, v_ref, o_ref):
        qm = q_ref[0] * 0.125
        km = k_ref[0]
        s = lax.dot_general(
            qm, km, (((1,), (1,)), ((), ())),
            preferred_element_type=jnp.float32,
        )
        i = lax.broadcasted_iota(jnp.int32, (SQ, SQ), 0)
        j = lax.broadcasted_iota(jnp.int32, (SQ, SQ), 1)
        mask = (jnp.abs(i - j) <= WINDOW).astype(jnp.bfloat16)
        w = jnp.exp(s.astype(jnp.bfloat16)) * mask
        denom = jnp.sum(w.astype(jnp.float32), axis=-1, keepdims=True)
        w = w * (1.0 / denom).astype(jnp.bfloat16)
        o_ref[0] = lax.dot_general(
            w, v_ref[0], (((1,), (0,)), ((), ())),
            preferred_element_type=jnp.float32,
        ).astype(jnp.bfloat16)

    spec = pl.BlockSpec((1, SQ, DH), lambda i: (i, 0, 0))
    return pl.pallas_call(
        body,
        grid=(n,),
        in_specs=[spec, spec, spec],
        out_specs=spec,
        out_shape=jax.ShapeDtypeStruct((n, SQ, DH), jnp.bfloat16),
    )(q, k, v)


def kernel(x, Wq, K_ext, V_ext, Wo):
    my = lax.axis_index("i")

    x_bf = x.astype(jnp.bfloat16)
    ag_q, ag_o = _allgather_weights(
        Wq.astype(jnp.bfloat16), Wo.astype(jnp.bfloat16)
    )

    perm = jnp.array([0, 3, 2, 1])[(jnp.arange(N_DEV) - my) % N_DEV]
    wq_g = jnp.take(ag_q, perm, axis=0)
    wo_g = jnp.take(ag_o, perm, axis=0)

    k = lax.dynamic_slice_in_dim(K_ext, my * B_PER, B_PER, axis=0)
    v = lax.dynamic_slice_in_dim(V_ext, my * B_PER, B_PER, axis=0)
    k = k.astype(jnp.bfloat16).transpose(0, 2, 1, 3).reshape(B_PER * HQ, SQ, DH)
    v = v.astype(jnp.bfloat16).transpose(0, 2, 1, 3).reshape(B_PER * HQ, SQ, DH)

    r = HQ // N_DEV
    wq5 = wq_g.reshape(N_DEV, x.shape[-1], r, DH)
    q = jnp.einsum("bsm,gmrd->bgrsd", x_bf, wq5,
                   preferred_element_type=jnp.bfloat16)
    q = q.reshape(B_PER * HQ, SQ, DH)

    ctx = _attention(q, k, v)
    ctx = ctx.reshape(B_PER, N_DEV, r, SQ, DH)

    wo5 = wo_g.reshape(N_DEV, r, DH, Wo.shape[-1])
    out = jnp.einsum("bgrsd,grde->bse", ctx, wo5,
                     preferred_element_type=jnp.float32)
    return out.astype(jnp.float32)


# device time: 135703 ns/iter; 1.1676x vs baseline; 1.0013x over previous
import jax
import jax.numpy as jnp
from jax import lax
from jax.experimental import pallas as pl
from jax.experimental.pallas import tpu as pltpu

N_DEV = 4
B_PER = 2
SQ = 512
HQ = 32
DH = 64
WINDOW = 128


def _allgather_weights(wq, wo):
    dq, fq = wq.shape
    fo, do = wo.shape

    def body(wq_ref, wo_ref, outq_ref, outo_ref,
             send_q, recv_q, send_o, recv_o):
        my = lax.axis_index("i")
        left = lax.rem(my - 1 + N_DEV, N_DEV)
        right = lax.rem(my + 1, N_DEV)

        outq_ref[0] = wq_ref[...]
        outo_ref[0] = wo_ref[...]

        barrier_sem = pltpu.get_barrier_semaphore()
        for nbr in (left, right):
            pl.semaphore_signal(
                barrier_sem, inc=1,
                device_id=(nbr,), device_id_type=pl.DeviceIdType.MESH,
            )
        pl.semaphore_wait(barrier_sem, 2)

        def send(out_ref, s_sems, r_sems, src_slot, dst_slot, sem, dst):
            return pltpu.make_async_remote_copy(
                src_ref=out_ref.at[src_slot],
                dst_ref=out_ref.at[dst_slot],
                send_sem=s_sems.at[sem],
                recv_sem=r_sems.at[sem],
                device_id=(dst,),
                device_id_type=pl.DeviceIdType.MESH,
            )

        r0 = [
            send(outq_ref, send_q, recv_q, 0, 1, 0, right),
            send(outo_ref, send_o, recv_o, 0, 1, 0, right),
            send(outq_ref, send_q, recv_q, 0, 3, 2, left),
            send(outo_ref, send_o, recv_o, 0, 3, 2, left),
        ]
        for r in r0:
            r.start()
        for r in r0:
            r.wait()
        r1 = [
            send(outq_ref, send_q, recv_q, 1, 2, 1, right),
            send(outo_ref, send_o, recv_o, 1, 2, 1, right),
        ]
        for r in r1:
            r.start()
        for r in r1:
            r.wait()

    return pl.pallas_call(
        body,
        out_shape=(
            jax.ShapeDtypeStruct((N_DEV, dq, fq), wq.dtype),
            jax.ShapeDtypeStruct((N_DEV, fo, do), wo.dtype),
        ),
        in_specs=[
            pl.BlockSpec(memory_space=pltpu.VMEM),
            pl.BlockSpec(memory_space=pltpu.VMEM),
        ],
        out_specs=(
            pl.BlockSpec(memory_space=pltpu.VMEM),
            pl.BlockSpec(memory_space=pltpu.VMEM),
        ),
        scratch_shapes=[
            pltpu.SemaphoreType.DMA((N_DEV - 1,)),
            pltpu.SemaphoreType.DMA((N_DEV - 1,)),
            pltpu.SemaphoreType.DMA((N_DEV - 1,)),
            pltpu.SemaphoreType.DMA((N_DEV - 1,)),
        ],
        compiler_params=pltpu.CompilerParams(collective_id=0),
    )(wq, wo)


def _attention(q, k, v, mask):
    n = q.shape[0]

    def body(q_ref, k_ref, v_ref, m_ref, o_ref):
        qm = q_ref[0] * 0.125
        km = k_ref[0]
        s = lax.dot_general(
            qm, km, (((1,), (1,)), ((), ())),
            preferred_element_type=jnp.float32,
        )
        w = jnp.exp(s.astype(jnp.bfloat16)) * m_ref[...]
        denom = jnp.sum(w.astype(jnp.float32), axis=-1, keepdims=True)
        w = w * (1.0 / denom).astype(jnp.bfloat16)
        o_ref[0] = lax.dot_general(
            w, v_ref[0], (((1,), (0,)), ((), ())),
            preferred_element_type=jnp.float32,
        ).astype(jnp.bfloat16)

    spec = pl.BlockSpec((1, SQ, DH), lambda i: (i, 0, 0))
    return pl.pallas_call(
        body,
        grid=(n,),
        in_specs=[spec, spec, spec,
                  pl.BlockSpec((SQ, SQ), lambda i: (0, 0))],
        out_specs=spec,
        out_shape=jax.ShapeDtypeStruct((n, SQ, DH), jnp.bfloat16),
    )(q, k, v, mask)


def kernel(x, Wq, K_ext, V_ext, Wo):
    my = lax.axis_index("i")

    x_bf = x.astype(jnp.bfloat16)
    ag_q, ag_o = _allgather_weights(
        Wq.astype(jnp.bfloat16), Wo.astype(jnp.bfloat16)
    )

    perm = jnp.array([0, 3, 2, 1])[(jnp.arange(N_DEV) - my) % N_DEV]
    wq_g = jnp.take(ag_q, perm, axis=0)
    wo_g = jnp.take(ag_o, perm, axis=0)

    k = lax.dynamic_slice_in_dim(K_ext, my * B_PER, B_PER, axis=0)
    v = lax.dynamic_slice_in_dim(V_ext, my * B_PER, B_PER, axis=0)
    k = k.astype(jnp.bfloat16).transpose(0, 2, 1, 3).reshape(B_PER * HQ, SQ, DH)
    v = v.astype(jnp.bfloat16).transpose(0, 2, 1, 3).reshape(B_PER * HQ, SQ, DH)

    r = HQ // N_DEV
    wq5 = wq_g.reshape(N_DEV, x.shape[-1], r, DH)
    q = jnp.einsum("bsm,gmrd->bgrsd", x_bf, wq5,
                   preferred_element_type=jnp.bfloat16)
    q = q.reshape(B_PER * HQ, SQ, DH)

    idx = jnp.arange(SQ)
    mask = (jnp.abs(idx[:, None] - idx[None, :]) <= WINDOW).astype(jnp.bfloat16)
    ctx = _attention(q, k, v, mask)
    ctx = ctx.reshape(B_PER, N_DEV, r, SQ, DH)

    wo5 = wo_g.reshape(N_DEV, r, DH, Wo.shape[-1])
    out = jnp.einsum("bgrsd,grde->bse", ctx, wo5,
                     preferred_element_type=jnp.float32)
    return out.astype(jnp.float32)


# device time: 75668 ns/iter; 2.0940x vs baseline; 1.7934x over previous
import jax
import jax.numpy as jnp
from jax import lax
from jax.experimental import pallas as pl
from jax.experimental.pallas import tpu as pltpu

N_DEV = 4
B_PER = 2
SQ = 512
HQ = 32
R = HQ // N_DEV
DH = 64
DM = 768
WINDOW = 128

_SLOT_OFF = (0, 3, 2, 1)


def _fused(x, wq, wo, ks, vs, mask):
    def body(x_ref, wq_ref, wo_ref, k_ref, v_ref, m_ref, out_ref,
             commq, commo, ctx_sc, send_q, recv_q, send_o, recv_o):
        my = lax.axis_index("i")
        left = lax.rem(my - 1 + N_DEV, N_DEV)
        right = lax.rem(my + 1, N_DEV)

        commq[0] = wq_ref[...]
        commo[0] = wo_ref[...]

        barrier_sem = pltpu.get_barrier_semaphore()
        for nbr in (left, right):
            pl.semaphore_signal(
                barrier_sem, inc=1,
                device_id=(nbr,), device_id_type=pl.DeviceIdType.MESH,
            )
        pl.semaphore_wait(barrier_sem, 2)

        def rdma(ref, s_sems, r_sems, src_slot, dst_slot, sem, dst):
            return pltpu.make_async_remote_copy(
                src_ref=ref.at[src_slot],
                dst_ref=ref.at[dst_slot],
                send_sem=s_sems.at[sem],
                recv_sem=r_sems.at[sem],
                device_id=(dst,),
                device_id_type=pl.DeviceIdType.MESH,
            )

        def compute(slot, init):
            wq_s = commq[slot]
            wo_s = commo[slot]
            for b in range(B_PER):
                qb = lax.dot_general(
                    x_ref[b], wq_s, (((1,), (0,)), ((), ())),
                    preferred_element_type=jnp.float32,
                )
                qb = (qb * 0.125).astype(jnp.bfloat16)
                for r in range(R):
                    qr = qb[:, r * DH:(r + 1) * DH]
                    s = lax.dot_general(
                        qr, k_ref[b, slot, r], (((1,), (1,)), ((), ())),
                        preferred_element_type=jnp.float32,
                    )
                    w = jnp.exp(s.astype(jnp.bfloat16)) * m_ref[...]
                    denom = jnp.sum(w.astype(jnp.float32), axis=-1,
                                    keepdims=True)
                    w = w * (1.0 / denom).astype(jnp.bfloat16)
                    ctx_sc[:, r * DH:(r + 1) * DH] = lax.dot_general(
                        w, v_ref[b, slot, r], (((1,), (0,)), ((), ())),
                        preferred_element_type=jnp.float32,
                    ).astype(jnp.bfloat16)
                contrib = lax.dot_general(
                    ctx_sc[...], wo_s, (((1,), (0,)), ((), ())),
                    preferred_element_type=jnp.float32,
                )
                if init:
                    out_ref[b] = contrib
                else:
                    out_ref[b] = out_ref[b] + contrib

        r0 = [
            rdma(commq, send_q, recv_q, 0, 1, 0, right),
            rdma(commo, send_o, recv_o, 0, 1, 0, right),
            rdma(commq, send_q, recv_q, 0, 3, 2, left),
            rdma(commo, send_o, recv_o, 0, 3, 2, left),
        ]
        for r in r0:
            r.start()
        compute(0, init=True)
        for r in r0:
            r.wait()

        r1 = [
            rdma(commq, send_q, recv_q, 1, 2, 1, right),
            rdma(commo, send_o, recv_o, 1, 2, 1, right),
        ]
        for r in r1:
            r.start()
        compute(3, init=False)
        compute(1, init=False)
        for r in r1:
            r.wait()
        compute(2, init=False)

    return pl.pallas_call(
        body,
        out_shape=jax.ShapeDtypeStruct((B_PER, SQ, DM), jnp.float32),
        in_specs=[pl.BlockSpec(memory_space=pltpu.VMEM)] * 6,
        out_specs=pl.BlockSpec(memory_space=pltpu.VMEM),
        scratch_shapes=[
            pltpu.VMEM((N_DEV, DM, R * DH), jnp.bfloat16),
            pltpu.VMEM((N_DEV, R * DH, DM), jnp.bfloat16),
            pltpu.VMEM((SQ, R * DH), jnp.bfloat16),
            pltpu.SemaphoreType.DMA((3,)),
            pltpu.SemaphoreType.DMA((3,)),
            pltpu.SemaphoreType.DMA((3,)),
            pltpu.SemaphoreType.DMA((3,)),
        ],
        compiler_params=pltpu.CompilerParams(collective_id=0),
    )(x, wq, wo, ks, vs, mask)


def kernel(x, Wq, K_ext, V_ext, Wo):
    my = lax.axis_index("i")

    x_bf = x.astype(jnp.bfloat16)
    origin_by_slot = (my + jnp.array(_SLOT_OFF)) % N_DEV

    def prep(t):
        t = lax.dynamic_slice_in_dim(t, my * B_PER, B_PER, axis=0)
        t = t.astype(jnp.bfloat16).transpose(0, 2, 1, 3)
        t = t.reshape(B_PER, N_DEV, R, SQ, DH)
        return jnp.take(t, origin_by_slot, axis=1)

    ks = prep(K_ext)
    vs = prep(V_ext)

    idx = jnp.arange(SQ)
    mask = (jnp.abs(idx[:, None] - idx[None, :]) <= WINDOW).astype(jnp.bfloat16)

    return _fused(
        x_bf,
        Wq.astype(jnp.bfloat16),
        Wo.astype(jnp.bfloat16),
        ks, vs, mask,
    )
